# Initial kernel scaffold; baseline (speedup 1.0000x reference)
import jax
import jax.numpy as jnp
from jax.experimental import pallas as pl


def kernel(users, items, user_emb, item_emb, W1, b1, W2, b2, W3, b3):
    raise NotImplementedError("write your pallas kernel here")


# speedup vs baseline: 1.1600x; 1.1600x over previous
import functools

import jax
import jax.numpy as jnp
from jax import lax
from jax.experimental import pallas as pl
from jax.experimental.pallas import tpu as pltpu
from jax.experimental.pallas import tpu_sc as plsc

_BATCH = 16384
_EMB = 64

_info = plsc.get_sparse_core_info()
_NC, _NS = _info.num_cores, _info.num_subcores
_NW = _NC * _NS
_BPW = _BATCH // _NW


def _sc_gather_body(users_hbm, items_hbm, uemb_hbm, iemb_hbm,
                    out_u_hbm, out_v_hbm,
                    uidx_v, iidx_v, urows_v, irows_v, sem_u, sem_v):
  wid = lax.axis_index("s") * _NC + lax.axis_index("c")
  base = wid * _BPW
  pltpu.sync_copy(users_hbm.at[pl.ds(base, _BPW)], uidx_v)
  pltpu.sync_copy(items_hbm.at[pl.ds(base, _BPW)], iidx_v)
  cu = pltpu.async_copy(uemb_hbm.at[uidx_v], urows_v, sem_u)
  ci = pltpu.async_copy(iemb_hbm.at[iidx_v], irows_v, sem_v)
  cu.wait()
  ci.wait()
  pltpu.sync_copy(urows_v, out_u_hbm.at[pl.ds(base, _BPW)])
  pltpu.sync_copy(irows_v, out_v_hbm.at[pl.ds(base, _BPW)])


_sc_gather = pl.kernel(
    _sc_gather_body,
    out_type=(
        jax.ShapeDtypeStruct((_BATCH, _EMB), jnp.float32),
        jax.ShapeDtypeStruct((_BATCH, _EMB), jnp.float32),
    ),
    mesh=plsc.VectorSubcoreMesh(core_axis_name="c", subcore_axis_name="s"),
    scratch_types=[
        pltpu.VMEM((_BPW,), jnp.int32),
        pltpu.VMEM((_BPW,), jnp.int32),
        pltpu.VMEM((_BPW, _EMB), jnp.float32),
        pltpu.VMEM((_BPW, _EMB), jnp.float32),
        pltpu.SemaphoreType.DMA,
        pltpu.SemaphoreType.DMA,
    ],
    compiler_params=pltpu.CompilerParams(use_tc_tiling_on_sc=False),
)


def _mlp_body(u_ref, v_ref, w1u_ref, w1v_ref, b1_ref, w2_ref, b2_ref,
              w3_ref, b3_ref, out_ref):
  x1 = (jnp.dot(u_ref[...], w1u_ref[...], preferred_element_type=jnp.float32)
        + jnp.dot(v_ref[...], w1v_ref[...], preferred_element_type=jnp.float32)
        + b1_ref[...])
  h1 = jnp.maximum(x1, 0.0)
  h2 = jnp.maximum(
      jnp.dot(h1, w2_ref[...], preferred_element_type=jnp.float32)
      + b2_ref[...], 0.0)
  out_ref[...] = (
      jnp.dot(h2, w3_ref[...], preferred_element_type=jnp.float32)
      + b3_ref[0, 0])


_MLP_BB = 2048


def _mlp_call(u, v, w1u, w1v, b1, w2t, b2, w3t, b3):
  grid = (_BATCH // _MLP_BB,)
  return pl.pallas_call(
      _mlp_body,
      grid=grid,
      in_specs=[
          pl.BlockSpec((_MLP_BB, _EMB), lambda i: (i, 0)),
          pl.BlockSpec((_MLP_BB, _EMB), lambda i: (i, 0)),
          pl.BlockSpec(w1u.shape, lambda i: (0, 0)),
          pl.BlockSpec(w1v.shape, lambda i: (0, 0)),
          pl.BlockSpec(b1.shape, lambda i: (0, 0)),
          pl.BlockSpec(w2t.shape, lambda i: (0, 0)),
          pl.BlockSpec(b2.shape, lambda i: (0, 0)),
          pl.BlockSpec(w3t.shape, lambda i: (0, 0)),
          pl.BlockSpec(b3.shape, lambda i: (0, 0)),
      ],
      out_specs=pl.BlockSpec((_MLP_BB, 1), lambda i: (i, 0)),
      out_shape=jax.ShapeDtypeStruct((_BATCH, 1), jnp.float32),
  )(u, v, w1u, w1v, b1, w2t, b2, w3t, b3)


@jax.jit
def kernel(users, items, user_emb, item_emb, W1, b1, W2, b2, W3, b3):
  u, v = _sc_gather(users, items, user_emb, item_emb)
  w1t = W1.T
  w1u = w1t[:_EMB]
  w1v = w1t[_EMB:]
  out = _mlp_call(u, v, w1u, w1v, b1.reshape(1, -1), W2.T,
                  b2.reshape(1, -1), W3.T, b3.reshape(1, 1))
  return out[:, 0]


# speedup vs baseline: 1.3076x; 1.1272x over previous
import functools

import jax
import jax.numpy as jnp
from jax import lax
from jax.experimental import pallas as pl
from jax.experimental.pallas import tpu as pltpu
from jax.experimental.pallas import tpu_sc as plsc

_BATCH = 16384
_EMB = 64

_info = plsc.get_sparse_core_info()
_NC, _NS = _info.num_cores, _info.num_subcores
_NW = _NC * _NS
_BPW = _BATCH // _NW


def _sc_gather_body(users_hbm, items_hbm, uemb_hbm, iemb_hbm,
                    out_u_hbm, out_v_hbm,
                    uidx_v, iidx_v, urows_v, irows_v, sem_u, sem_v):
  wid = lax.axis_index("s") * _NC + lax.axis_index("c")
  base = wid * _BPW
  pltpu.sync_copy(users_hbm.at[pl.ds(base, _BPW)], uidx_v)
  pltpu.sync_copy(items_hbm.at[pl.ds(base, _BPW)], iidx_v)
  cu = pltpu.async_copy(uemb_hbm.at[uidx_v], urows_v, sem_u)
  ci = pltpu.async_copy(iemb_hbm.at[iidx_v], irows_v, sem_v)
  cu.wait()
  ci.wait()
  pltpu.sync_copy(urows_v, out_u_hbm.at[pl.ds(base, _BPW)])
  pltpu.sync_copy(irows_v, out_v_hbm.at[pl.ds(base, _BPW)])


_sc_gather = pl.kernel(
    _sc_gather_body,
    out_type=(
        jax.ShapeDtypeStruct((_BATCH, _EMB), jnp.float32),
        jax.ShapeDtypeStruct((_BATCH, _EMB), jnp.float32),
    ),
    mesh=plsc.VectorSubcoreMesh(core_axis_name="c", subcore_axis_name="s"),
    scratch_types=[
        pltpu.VMEM((_BPW,), jnp.int32),
        pltpu.VMEM((_BPW,), jnp.int32),
        pltpu.VMEM((_BPW, _EMB), jnp.float32),
        pltpu.VMEM((_BPW, _EMB), jnp.float32),
        pltpu.SemaphoreType.DMA,
        pltpu.SemaphoreType.DMA,
    ],
    compiler_params=pltpu.CompilerParams(use_tc_tiling_on_sc=False),
)


def _mlp_body(u2_ref, v2_ref, w1u_ref, w1v_ref, b1_ref, w2_ref, b2_ref,
              w3_ref, b3_ref, out_ref):
  ue = u2_ref[:, :_EMB]
  uo = u2_ref[:, _EMB:]
  ve = v2_ref[:, :_EMB]
  vo = v2_ref[:, _EMB:]

  def mlp(uu, vv):
    x1 = (jnp.dot(uu, w1u_ref[...], preferred_element_type=jnp.float32)
          + jnp.dot(vv, w1v_ref[...], preferred_element_type=jnp.float32)
          + b1_ref[...])
    h1 = jnp.maximum(x1, 0.0)
    h2 = jnp.maximum(
        jnp.dot(h1, w2_ref[...], preferred_element_type=jnp.float32)
        + b2_ref[...], 0.0)
    return (jnp.dot(h2, w3_ref[...], preferred_element_type=jnp.float32)
            + b3_ref[0, 0])

  oe = mlp(ue, ve)
  oo = mlp(uo, vo)
  out_ref[...] = jnp.concatenate([oe, oo], axis=1)


_MLP_BB = 2048


def _mlp_call(u2, v2, w1u, w1v, b1, w2t, b2, w3t, b3):
  grid = ((_BATCH // 2) // _MLP_BB,)
  return pl.pallas_call(
      _mlp_body,
      grid=grid,
      in_specs=[
          pl.BlockSpec((_MLP_BB, 2 * _EMB), lambda i: (i, 0)),
          pl.BlockSpec((_MLP_BB, 2 * _EMB), lambda i: (i, 0)),
          pl.BlockSpec(w1u.shape, lambda i: (0, 0)),
          pl.BlockSpec(w1v.shape, lambda i: (0, 0)),
          pl.BlockSpec(b1.shape, lambda i: (0, 0)),
          pl.BlockSpec(w2t.shape, lambda i: (0, 0)),
          pl.BlockSpec(b2.shape, lambda i: (0, 0)),
          pl.BlockSpec(w3t.shape, lambda i: (0, 0)),
          pl.BlockSpec(b3.shape, lambda i: (0, 0)),
      ],
      out_specs=pl.BlockSpec((_MLP_BB, 2), lambda i: (i, 0)),
      out_shape=jax.ShapeDtypeStruct((_BATCH // 2, 2), jnp.float32),
  )(u2, v2, w1u, w1v, b1, w2t, b2, w3t, b3)


@jax.jit
def kernel(users, items, user_emb, item_emb, W1, b1, W2, b2, W3, b3):
  u, v = _sc_gather(users, items, user_emb, item_emb)
  u2 = u.reshape(_BATCH // 2, 2 * _EMB)
  v2 = v.reshape(_BATCH // 2, 2 * _EMB)
  w1t = W1.T
  w1u = w1t[:_EMB]
  w1v = w1t[_EMB:]
  out = _mlp_call(u2, v2, w1u, w1v, b1.reshape(1, -1), W2.T,
                  b2.reshape(1, -1), W3.T, b3.reshape(1, 1))
  return out.reshape(_BATCH)


# speedup vs baseline: 1.6378x; 1.2526x over previous
import functools

import jax
import jax.numpy as jnp
from jax import lax
from jax.experimental import pallas as pl
from jax.experimental.pallas import tpu as pltpu
from jax.experimental.pallas import tpu_sc as plsc

_BATCH = 16384
_EMB = 64

_info = plsc.get_sparse_core_info()
_NC, _NS = _info.num_cores, _info.num_subcores
_NW = _NC * _NS
_BPW = _BATCH // _NW
_HALF = _BPW // 2


def _sc_gather_body(users_hbm, items_hbm, uemb_hbm, iemb_hbm,
                    out_u_hbm, out_v_hbm,
                    uidx_v, iidx_v, urows_v, irows_v, sem_u, sem_v):
  wid = lax.axis_index("s") * _NC + lax.axis_index("c")
  base = wid * _BPW
  pltpu.sync_copy(users_hbm.at[pl.ds(base, _BPW)], uidx_v)
  pltpu.sync_copy(items_hbm.at[pl.ds(base, _BPW)], iidx_v)

  def half(h, _):
    hbase = h * _HALF

    def lp(c, _):
      cu = uidx_v[pl.ds(hbase + c * 16, 16)]
      ci = iidx_v[pl.ds(hbase + c * 16, 16)]
      for k in range(16):
        j = c * 16 + k
        pltpu.async_copy(uemb_hbm.at[pl.ds(cu[k], 1)],
                         urows_v.at[pl.ds(j, 1)], sem_u)
        pltpu.async_copy(iemb_hbm.at[pl.ds(ci[k], 1)],
                         irows_v.at[pl.ds(j, 1)], sem_v)
      return 0

    lax.fori_loop(0, _HALF // 16, lp, 0)
    obase = base + hbase
    pltpu.make_async_copy(out_u_hbm.at[pl.ds(obase, _HALF)], urows_v,
                          sem_u).wait()
    pltpu.make_async_copy(out_v_hbm.at[pl.ds(obase, _HALF)], irows_v,
                          sem_v).wait()
    pltpu.sync_copy(urows_v, out_u_hbm.at[pl.ds(obase, _HALF)])
    pltpu.sync_copy(irows_v, out_v_hbm.at[pl.ds(obase, _HALF)])
    return 0

  lax.fori_loop(0, 2, half, 0)


_sc_gather = pl.kernel(
    _sc_gather_body,
    out_type=(
        jax.ShapeDtypeStruct((_BATCH, _EMB), jnp.float32),
        jax.ShapeDtypeStruct((_BATCH, _EMB), jnp.float32),
    ),
    mesh=plsc.VectorSubcoreMesh(core_axis_name="c", subcore_axis_name="s"),
    scratch_types=[
        pltpu.VMEM((_BPW,), jnp.int32),
        pltpu.VMEM((_BPW,), jnp.int32),
        pltpu.VMEM((_HALF, _EMB), jnp.float32),
        pltpu.VMEM((_HALF, _EMB), jnp.float32),
        pltpu.SemaphoreType.DMA,
        pltpu.SemaphoreType.DMA,
    ],
)


def _mlp_body(u_ref, v_ref, w1u_ref, w1v_ref, b1_ref, w2_ref, b2_ref,
              w3_ref, b3_ref, out_ref):
  x1 = (jnp.dot(u_ref[...], w1u_ref[...], preferred_element_type=jnp.float32)
        + jnp.dot(v_ref[...], w1v_ref[...], preferred_element_type=jnp.float32)
        + b1_ref[...])
  h1 = jnp.maximum(x1, 0.0)
  h2 = jnp.maximum(
      jnp.dot(h1, w2_ref[...], preferred_element_type=jnp.float32)
      + b2_ref[...], 0.0)
  out_ref[...] = (
      jnp.dot(h2, w3_ref[...], preferred_element_type=jnp.float32)
      + b3_ref[0, 0])


_MLP_BB = 2048


def _mlp_call(u, v, w1u, w1v, b1, w2t, b2, w3t, b3):
  grid = (_BATCH // _MLP_BB,)
  return pl.pallas_call(
      _mlp_body,
      grid=grid,
      in_specs=[
          pl.BlockSpec((_MLP_BB, _EMB), lambda i: (i, 0)),
          pl.BlockSpec((_MLP_BB, _EMB), lambda i: (i, 0)),
          pl.BlockSpec(w1u.shape, lambda i: (0, 0)),
          pl.BlockSpec(w1v.shape, lambda i: (0, 0)),
          pl.BlockSpec(b1.shape, lambda i: (0, 0)),
          pl.BlockSpec(w2t.shape, lambda i: (0, 0)),
          pl.BlockSpec(b2.shape, lambda i: (0, 0)),
          pl.BlockSpec(w3t.shape, lambda i: (0, 0)),
          pl.BlockSpec(b3.shape, lambda i: (0, 0)),
      ],
      out_specs=pl.BlockSpec((_MLP_BB, 1), lambda i: (i, 0)),
      out_shape=jax.ShapeDtypeStruct((_BATCH, 1), jnp.float32),
  )(u, v, w1u, w1v, b1, w2t, b2, w3t, b3)


@jax.jit
def kernel(users, items, user_emb, item_emb, W1, b1, W2, b2, W3, b3):
  u, v = _sc_gather(users, items, user_emb, item_emb)
  w1t = W1.T
  w1u = w1t[:_EMB]
  w1v = w1t[_EMB:]
  out = _mlp_call(u, v, w1u, w1v, b1.reshape(1, -1), W2.T,
                  b2.reshape(1, -1), W3.T, b3.reshape(1, 1))
  return out[:, 0]
